# baseline (device time: 177352 ns/iter reference)
import jax
import jax.numpy as jnp
from jax import lax
from jax.experimental import pallas as pl
from jax.experimental.pallas import tpu as pltpu

N_Y = 4
N_CHUNK = 8
N_STORE = 8
_MESH = pl.DeviceIdType.MESH


def kernel(x):
    m, n = x.shape
    half = m // 2
    ch = half // N_CHUNK
    M = N_Y * m

    def body(
        x_hbm,
        out_hbm,
        comm,
        stage,
        ysend_sems,
        yrecv_sems,
        xsend_sems,
        xrecv_sems,
        load_sems,
        store_sems,
    ):
        my_x = lax.axis_index("x")
        my_y = lax.axis_index("y")
        my_z = lax.axis_index("z")
        h_off = my_x * half
        o_off = half - h_off
        partner = (1 - my_x, my_y, my_z)

        def mychunk(s, c):
            return comm.at[pl.ds(s * m + h_off + c * ch, ch), :]

        def othchunk(s, c):
            return comm.at[pl.ds(s * m + o_off + c * ch, ch), :]

        barrier = pltpu.get_barrier_semaphore()

        for Y in range(N_Y):

            @pl.when(my_y == Y)
            def _(Y=Y):
                nbrs = [partner]
                if Y > 0:
                    nbrs.append((my_x, Y - 1, my_z))
                if Y < N_Y - 1:
                    nbrs.append((my_x, Y + 1, my_z))
                for nbr in nbrs:
                    pl.semaphore_signal(
                        barrier, inc=1, device_id=nbr, device_id_type=_MESH
                    )
                pl.semaphore_wait(barrier, len(nbrs))

                sends = []
                stores = []

                def y_send(s, c, to_y, dirn):
                    cp = pltpu.make_async_remote_copy(
                        src_ref=mychunk(s, c),
                        dst_ref=mychunk(s, c),
                        send_sem=ysend_sems.at[s, dirn, c],
                        recv_sem=yrecv_sems.at[s, c],
                        device_id=(my_x, to_y, my_z),
                        device_id_type=_MESH,
                    )
                    cp.start()
                    sends.append(cp)

                def x_send(s, c):
                    cp = pltpu.make_async_remote_copy(
                        src_ref=mychunk(s, c),
                        dst_ref=mychunk(s, c),
                        send_sem=xsend_sems.at[s, c],
                        recv_sem=xrecv_sems.at[s, c],
                        device_id=partner,
                        device_id_type=_MESH,
                    )
                    cp.start()
                    sends.append(cp)

                def y_wait(s, c):
                    r = pltpu.make_async_remote_copy(
                        src_ref=mychunk(s, c),
                        dst_ref=mychunk(s, c),
                        send_sem=ysend_sems.at[s, 0, c],
                        recv_sem=yrecv_sems.at[s, c],
                        device_id=partner,
                        device_id_type=_MESH,
                    )
                    r.wait_recv()

                def store_out(row):
                    if len(stores) >= N_STORE:
                        stores[len(stores) - N_STORE].wait()
                    cp = pltpu.make_async_copy(
                        comm.at[pl.ds(row, ch), :],
                        out_hbm.at[pl.ds(row, ch), :],
                        store_sems.at[len(stores) % N_STORE],
                    )
                    cp.start()
                    stores.append(cp)

                def stream_own(rows, inject):
                    loads = []
                    for i in range(min(2, len(rows))):
                        cp = pltpu.make_async_copy(
                            x_hbm.at[pl.ds(rows[i], ch), :],
                            stage.at[i % 2],
                            load_sems.at[i % 2],
                        )
                        cp.start()
                        loads.append(cp)
                    for i, row in enumerate(rows):
                        loads[i].wait()
                        comm[pl.ds(Y * m + row, ch), :] = stage[
                            i % 2
                        ].astype(comm.dtype)
                        if i + 2 < len(rows):
                            cp = pltpu.make_async_copy(
                                x_hbm.at[pl.ds(rows[i + 2], ch), :],
                                stage.at[i % 2],
                                load_sems.at[i % 2],
                            )
                            cp.start()
                            loads.append(cp)
                        if inject:
                            if Y < N_Y - 1:
                                y_send(Y, i, Y + 1, 1)
                            if Y > 0:
                                y_send(Y, i, Y - 1, 0)
                        store_out(Y * m + row)

                stream_own(
                    [h_off + c * ch for c in range(N_CHUNK)], inject=True
                )

                for d in (1, 2, 3):
                    sl, sr = Y - d, Y + d
                    if sl >= 0:
                        for c in range(N_CHUNK):
                            y_wait(sl, c)
                            if Y < N_Y - 1:
                                y_send(sl, c, Y + 1, 1)
                            x_send(sl, c)
                            store_out(sl * m + h_off + c * ch)
                    if sr <= N_Y - 1:
                        for c in range(N_CHUNK):
                            y_wait(sr, c)
                            if Y > 0:
                                y_send(sr, c, Y - 1, 0)
                            x_send(sr, c)
                            store_out(sr * m + h_off + c * ch)

                stream_own(
                    [o_off + c * ch for c in range(N_CHUNK)], inject=False
                )

                for s in range(N_Y):
                    if s != Y:
                        for c in range(N_CHUNK):
                            r = pltpu.make_async_remote_copy(
                                src_ref=othchunk(s, c),
                                dst_ref=othchunk(s, c),
                                send_sem=xsend_sems.at[s, c],
                                recv_sem=xrecv_sems.at[s, c],
                                device_id=partner,
                                device_id_type=_MESH,
                            )
                            r.wait_recv()
                            store_out(s * m + o_off + c * ch)

                for cp in sends:
                    cp.wait_send()
                for cp in stores[max(0, len(stores) - N_STORE):]:
                    cp.wait()

    out = pl.pallas_call(
        body,
        out_shape=jax.ShapeDtypeStruct((M, n), jnp.bfloat16),
        in_specs=[pl.BlockSpec(memory_space=pl.ANY)],
        out_specs=pl.BlockSpec(memory_space=pl.ANY),
        scratch_shapes=[
            pltpu.VMEM((M, n), jnp.bfloat16),
            pltpu.VMEM((2, ch, n), jnp.float32),
            pltpu.SemaphoreType.DMA((N_Y, 2, N_CHUNK)),
            pltpu.SemaphoreType.DMA((N_Y, N_CHUNK)),
            pltpu.SemaphoreType.DMA((N_Y, N_CHUNK)),
            pltpu.SemaphoreType.DMA((N_Y, N_CHUNK)),
            pltpu.SemaphoreType.DMA((2,)),
            pltpu.SemaphoreType.DMA((N_STORE,)),
        ],
        compiler_params=pltpu.CompilerParams(
            collective_id=1, vmem_limit_bytes=48 * 1024 * 1024
        ),
    )(x)
    return lax.optimization_barrier(out)
